# baseline (speedup 1.0000x reference)
import functools

import jax
import jax.numpy as jnp
import numpy as np
from jax import lax
from jax.experimental import pallas as pl
from jax.experimental.pallas import tpu as pltpu
from jax.experimental.pallas import tpu_sc as plsc

NUM_GRIDS = 32
INPUT_DIM = 64
HIDDEN_DIM = 32
OUTPUT_DIM = 64
BATCH = 4096

NC = 2
NS = 16
NW = NC * NS
BPW = BATCH // NW
NCHUNK = BPW // 16

P_IN = INPUT_DIM // 2
P_H = HIDDEN_DIM // 2

T0_PX = 33 * 33
T1_PZ = 33 * 33
T1_PX = P_H * T1_PZ


def _get_borders(n):
    def inv(x):
        return np.log(2.0 * x) if x <= 0.5 else -np.log(2.0 * (1.0 - x))
    cs = 1.0 / n
    b = [inv(i * cs) for i in range(1, n)]
    left = b[0] - (b[1] - b[0])
    right = b[-1] + (b[-1] - b[-2])
    return np.array([left] + b + [right], dtype=np.float32)


_BORDERS = _get_borders(NUM_GRIDS)
_INV_LEN = (1.0 / (_BORDERS[1:] - _BORDERS[:-1])).astype(np.float32)

_BT = np.zeros((96,), np.float32)
_BT[:33] = _BORDERS
_BT[48:80] = _INV_LEN


def _sc_body(x_hbm, t0_hbm, t1_hbm, bt_hbm, out_hbm,
             xbuf, btbuf, s1idx, s1w, idx1buf,
             s1rowsA, s1rowsB, z0buf, jzbuf, cz0buf, cz1buf,
             s3idxA, s3idxB, s3wA, s3wB, s3rowsA, s3rowsB,
             outacc, outfix, semA, semB):
    wid = lax.axis_index("s") * NC + lax.axis_index("c")
    lanes = jnp.arange(16, dtype=jnp.int32)
    zeros16 = jnp.zeros((16,), jnp.float32)

    pltpu.sync_copy(x_hbm.at[:, pl.ds(wid * BPW, BPW)], xbuf)
    pltpu.sync_copy(bt_hbm, btbuf)

    def interp(v):
        e = jnp.exp(-jnp.abs(v))
        cdf = jnp.where(v > 0.0, 1.0 - 0.5 * e, 0.5 * e)
        idx = (cdf * float(NUM_GRIDS)).astype(jnp.int32)
        idx = jnp.minimum(jnp.maximum(idx, 0), NUM_GRIDS - 1)
        b = plsc.load_gather(btbuf, [idx])
        il = plsc.load_gather(btbuf, [48 + idx])
        return idx, (v - b) * il

    def wsplat(ref, i):
        return plsc.load_gather(ref, [jnp.full((16,), i, jnp.int32)])

    def chunk_body(c, _):
        b0 = c * 16

        def zero_body(b, _):
            z0buf[pl.ds(b * 32, 16)] = zeros16
            z0buf[pl.ds(b * 32 + 16, 16)] = zeros16
            for q in range(4):
                outacc[b, pl.ds(q * 16, 16)] = zeros16
            return _
        lax.fori_loop(0, 16, zero_body, None)

        def s1gen(px, _):
            xv1 = xbuf[2 * px, pl.ds(b0, 16)]
            xv2 = xbuf[2 * px + 1, pl.ds(b0, 16)]
            ia, da = interp(xv1)
            ib, db = interp(xv2)
            base = px * T0_PX + ia * 33 + ib
            o = px * 64
            s1idx[pl.ds(o, 16)] = base
            s1idx[pl.ds(o + 16, 16)] = base + 1
            s1idx[pl.ds(o + 32, 16)] = base + 33
            s1idx[pl.ds(o + 48, 16)] = base + 34
            wa = 1.0 - da
            wb = 1.0 - db
            s1w[pl.ds(o, 16)] = wa * wb
            s1w[pl.ds(o + 16, 16)] = wa * db
            s1w[pl.ds(o + 32, 16)] = da * wb
            s1w[pl.ds(o + 48, 16)] = da * db
            idx1buf[px, :] = ia
            return _
        lax.fori_loop(0, P_IN, s1gen, None)

        def s1fire(q, rows, sem):
            for j in range(4):
                pltpu.async_copy(
                    t0_hbm.at[s1idx.at[pl.ds(q * 512 + j * 128, 128)]],
                    rows.at[pl.ds(j * 128, 128), :], sem)

        def s1drain(rows, sem):
            for j in range(4):
                pltpu.make_async_copy(
                    t0_hbm.at[s1idx.at[pl.ds(j * 128, 128)]],
                    rows.at[pl.ds(j * 128, 128), :], sem).wait()

        def s1acc(q, rows):
            def body(b, _):
                a0 = z0buf[pl.ds(b * 32, 16)]
                a1 = z0buf[pl.ds(b * 32 + 16, 16)]

                def inner(k, accs):
                    p0, p1 = accs
                    r = k * 16 + b
                    w = wsplat(s1w, q * 512 + r)
                    p0 = p0 + w * rows[r, pl.ds(0, 16)]
                    p1 = p1 + w * rows[r, pl.ds(16, 16)]
                    return (p0, p1)
                a0, a1 = lax.fori_loop(0, 32, inner, (a0, a1))
                z0buf[pl.ds(b * 32, 16)] = a0
                z0buf[pl.ds(b * 32 + 16, 16)] = a1
                return _
            lax.fori_loop(0, 16, body, None)

        s1fire(0, s1rowsA, semA)
        s1fire(1, s1rowsB, semB)
        s1drain(s1rowsA, semA)
        s1acc(0, s1rowsA)
        s1fire(2, s1rowsA, semA)
        s1drain(s1rowsB, semB)
        s1acc(1, s1rowsB)
        s1fire(3, s1rowsB, semB)
        s1drain(s1rowsA, semA)
        s1acc(2, s1rowsA)
        s1drain(s1rowsB, semB)
        s1acc(3, s1rowsB)

        def s2gen(pz, _):
            z1 = plsc.load_gather(z0buf, [lanes * 32 + 2 * pz])
            z2 = plsc.load_gather(z0buf, [lanes * 32 + 2 * pz + 1])
            iz1, dz1 = interp(z1)
            iz2_unused, dz2 = interp(z2)
            del iz2_unused
            jzbuf[pz, :] = iz1
            f = 1.0 - dz2
            cz0buf[pz, :] = (1.0 - dz1) * f
            cz1buf[pz, :] = dz1 * f
            return _
        lax.fori_loop(0, P_H, s2gen, None)

        def s3genfire(blk, idxr, wr, rows, sem):
            px = lax.shift_right_logical(blk, 1) & (P_IN - 1)
            half = blk & 1
            ia = idx1buf[px, :]
            a0 = s1w[pl.ds(px * 64, 16)]
            a1 = s1w[pl.ds(px * 64 + 32, 16)]
            pbase = px * T1_PX + ia * 33

            def gen(pzl, _):
                pz = half * 8 + pzl
                jz = jzbuf[pz, :]
                c0 = cz0buf[pz, :]
                c1 = cz1buf[pz, :]
                base = pbase + pz * T1_PZ + jz
                o = pzl * 32
                idxr[pl.ds(o, 16)] = base
                idxr[pl.ds(o + 16, 16)] = base + 33
                o = pzl * 64
                wr[pl.ds(o, 16)] = a0 * c0
                wr[pl.ds(o + 16, 16)] = a0 * c1
                wr[pl.ds(o + 32, 16)] = a1 * c0
                wr[pl.ds(o + 48, 16)] = a1 * c1
                return _
            lax.fori_loop(0, 8, gen, None)
            for j in range(2):
                pltpu.async_copy(
                    t1_hbm.at[idxr.at[pl.ds(j * 128, 128)]],
                    rows.at[pl.ds(j * 128, 128), :], sem)

        def s3drain(idxr, rows, sem):
            for j in range(2):
                pltpu.make_async_copy(
                    t1_hbm.at[idxr.at[pl.ds(j * 128, 128)]],
                    rows.at[pl.ds(j * 128, 128), :], sem).wait()

        def s3fma(wr, rows):
            def body(b, _):
                acc = [outacc[b, pl.ds(q * 16, 16)] for q in range(4)]
                for m in range(16):
                    row = m * 16 + b
                    w0 = wsplat(wr, 32 * m + b)
                    w1 = wsplat(wr, 32 * m + 16 + b)
                    for jc, w in ((0, w0), (1, w1)):
                        pa = plsc.bitcast(rows[row, pl.ds(jc * 64, 32)], jnp.int32)
                        pb = plsc.bitcast(rows[row, pl.ds(jc * 64 + 32, 32)], jnp.int32)
                        acc[0] = acc[0] + w * plsc.bitcast(lax.shift_left(pa, 16), jnp.float32)
                        acc[1] = acc[1] + w * plsc.bitcast(pa, jnp.float32)
                        acc[2] = acc[2] + w * plsc.bitcast(lax.shift_left(pb, 16), jnp.float32)
                        acc[3] = acc[3] + w * plsc.bitcast(pb, jnp.float32)
                for q in range(4):
                    outacc[b, pl.ds(q * 16, 16)] = acc[q]
                return _
            lax.fori_loop(0, 16, body, None)

        s3genfire(jnp.int32(0), s3idxA, s3wA, s3rowsA, semA)

        def s3pair(g, _):
            blk = g * 2
            s3genfire(blk + 1, s3idxB, s3wB, s3rowsB, semB)
            s3drain(s3idxA, s3rowsA, semA)
            s3fma(s3wA, s3rowsA)
            s3genfire(blk + 2, s3idxA, s3wA, s3rowsA, semA)
            s3drain(s3idxB, s3rowsB, semB)
            s3fma(s3wB, s3rowsB)
            return _
        lax.fori_loop(0, P_IN, s3pair, None)
        s3drain(s3idxA, s3rowsA, semA)

        def fixup(b, _):
            for q in range(4):
                pos = b * 64 + (q // 2) * 32 + (q & 1) + 2 * lanes
                plsc.store_scatter(outfix, [pos], outacc[b, pl.ds(q * 16, 16)])
            return _
        lax.fori_loop(0, 16, fixup, None)
        pltpu.sync_copy(outfix, out_hbm.at[pl.ds((wid * BPW + b0) * 64, 1024)])
        return _
    lax.fori_loop(0, NCHUNK, chunk_body, None)


@jax.jit
def _run(x, t0, t1, bt):
    mesh = plsc.VectorSubcoreMesh(core_axis_name="c", subcore_axis_name="s")
    f = functools.partial(
        pl.kernel,
        out_type=jax.ShapeDtypeStruct((BATCH * OUTPUT_DIM,), jnp.float32),
        mesh=mesh,
        compiler_params=pltpu.CompilerParams(
            needs_layout_passes=False, use_tc_tiling_on_sc=False),
        scratch_types=[
            pltpu.VMEM((INPUT_DIM, BPW), jnp.float32),
            pltpu.VMEM((96,), jnp.float32),
            pltpu.VMEM((2048,), jnp.int32),
            pltpu.VMEM((2048,), jnp.float32),
            pltpu.VMEM((P_IN, 16), jnp.int32),
            pltpu.VMEM((512, HIDDEN_DIM), jnp.float32),
            pltpu.VMEM((512, HIDDEN_DIM), jnp.float32),
            pltpu.VMEM((16 * HIDDEN_DIM,), jnp.float32),
            pltpu.VMEM((P_H, 16), jnp.int32),
            pltpu.VMEM((P_H, 16), jnp.float32),
            pltpu.VMEM((P_H, 16), jnp.float32),
            pltpu.VMEM((256,), jnp.int32),
            pltpu.VMEM((256,), jnp.int32),
            pltpu.VMEM((512,), jnp.float32),
            pltpu.VMEM((512,), jnp.float32),
            pltpu.VMEM((256, 2 * OUTPUT_DIM), jnp.bfloat16),
            pltpu.VMEM((256, 2 * OUTPUT_DIM), jnp.bfloat16),
            pltpu.VMEM((16, OUTPUT_DIM), jnp.float32),
            pltpu.VMEM((16 * OUTPUT_DIM,), jnp.float32),
            pltpu.SemaphoreType.DMA,
            pltpu.SemaphoreType.DMA,
        ],
    )(_sc_body)
    return f(x, t0, t1, bt)


def kernel(x, level0_params, level1_params):
    t0 = jnp.transpose(level0_params, (3, 0, 1, 2)).reshape(P_IN * T0_PX, HIDDEN_DIM)
    t1f = jnp.transpose(level1_params, (3, 4, 0, 1, 2)).reshape(
        P_IN * T1_PX, OUTPUT_DIM).astype(jnp.bfloat16)
    t1n = jnp.concatenate(
        [t1f[1:], jnp.zeros((1, OUTPUT_DIM), jnp.bfloat16)], axis=0)
    t1 = jnp.concatenate([t1f, t1n], axis=1)
    bt = jnp.asarray(_BT)
    out_t = _run(x, t0, t1, bt)
    return out_t.reshape(BATCH, OUTPUT_DIM).T


# speedup vs baseline: 1.4305x; 1.4305x over previous
import functools

import jax
import jax.numpy as jnp
import numpy as np
from jax import lax
from jax.experimental import pallas as pl
from jax.experimental.pallas import tpu as pltpu
from jax.experimental.pallas import tpu_sc as plsc

NUM_GRIDS = 32
INPUT_DIM = 64
HIDDEN_DIM = 32
OUTPUT_DIM = 64
BATCH = 4096

NC = 2
NS = 16
NW = NC * NS
BPW = BATCH // NW
NCHUNK = BPW // 16

P_IN = INPUT_DIM // 2
P_H = HIDDEN_DIM // 2

T0_PX = 33 * 33
T1_PZ = 33 * 33
T1_PX = P_H * T1_PZ


def _get_borders(n):
    def inv(x):
        return np.log(2.0 * x) if x <= 0.5 else -np.log(2.0 * (1.0 - x))
    cs = 1.0 / n
    b = [inv(i * cs) for i in range(1, n)]
    left = b[0] - (b[1] - b[0])
    right = b[-1] + (b[-1] - b[-2])
    return np.array([left] + b + [right], dtype=np.float32)


_BORDERS = _get_borders(NUM_GRIDS)
_INV_LEN = (1.0 / (_BORDERS[1:] - _BORDERS[:-1])).astype(np.float32)

_BT = np.zeros((96,), np.float32)
_BT[:33] = _BORDERS
_BT[48:80] = _INV_LEN


def _sc_body(x_hbm, t0_hbm, t1_hbm, bt_hbm, out_hbm,
             xbuf, btbuf, s1idx, s1w, idx1buf,
             s1rowsA, s1rowsB, z0buf, jzbuf, cz0buf, cz1buf,
             s3idxA, s3idxB, s3wA, s3wB, s3rowsA, s3rowsB,
             outacc, outfix, semA, semB):
    wid = lax.axis_index("s") * NC + lax.axis_index("c")
    lanes = jnp.arange(16, dtype=jnp.int32)
    zeros16 = jnp.zeros((16,), jnp.float32)

    pltpu.sync_copy(x_hbm.at[:, pl.ds(wid * BPW, BPW)], xbuf)
    pltpu.sync_copy(bt_hbm, btbuf)

    def interp(v):
        e = jnp.exp(-jnp.abs(v))
        cdf = jnp.where(v > 0.0, 1.0 - 0.5 * e, 0.5 * e)
        idx = (cdf * float(NUM_GRIDS)).astype(jnp.int32)
        idx = jnp.minimum(jnp.maximum(idx, 0), NUM_GRIDS - 1)
        b = plsc.load_gather(btbuf, [idx])
        il = plsc.load_gather(btbuf, [48 + idx])
        return idx, (v - b) * il

    def wsplat(ref, i):
        return plsc.load_gather(ref, [jnp.full((16,), i, jnp.int32)])

    def chunk_body(c, _):
        b0 = c * 16

        def zero_body(b, _):
            z0buf[pl.ds(b * 32, 16)] = zeros16
            z0buf[pl.ds(b * 32 + 16, 16)] = zeros16
            for q in range(4):
                outacc[b, pl.ds(q * 16, 16)] = zeros16
            return _
        lax.fori_loop(0, 16, zero_body, None)

        def s1gen(px, _):
            xv1 = xbuf[2 * px, pl.ds(b0, 16)]
            xv2 = xbuf[2 * px + 1, pl.ds(b0, 16)]
            ia, da = interp(xv1)
            ib, db = interp(xv2)
            base = px * T0_PX + ia * 33 + ib
            o = px * 64
            s1idx[pl.ds(o, 16)] = base
            s1idx[pl.ds(o + 16, 16)] = base + 1
            s1idx[pl.ds(o + 32, 16)] = base + 33
            s1idx[pl.ds(o + 48, 16)] = base + 34
            wa = 1.0 - da
            wb = 1.0 - db
            s1w[pl.ds(o, 16)] = wa * wb
            s1w[pl.ds(o + 16, 16)] = wa * db
            s1w[pl.ds(o + 32, 16)] = da * wb
            s1w[pl.ds(o + 48, 16)] = da * db
            idx1buf[px, :] = ia
            return _
        lax.fori_loop(0, P_IN, s1gen, None)

        def s1fire(q, rows, sem):
            for j in range(4):
                pltpu.async_copy(
                    t0_hbm.at[s1idx.at[pl.ds(q * 512 + j * 128, 128)]],
                    rows.at[pl.ds(j * 128, 128), :], sem)

        def s1drain(rows, sem):
            for j in range(4):
                pltpu.make_async_copy(
                    t0_hbm.at[s1idx.at[pl.ds(j * 128, 128)]],
                    rows.at[pl.ds(j * 128, 128), :], sem).wait()

        def s1acc(q, rows):
            def body(b, _):
                a0 = z0buf[pl.ds(b * 32, 16)]
                a1 = z0buf[pl.ds(b * 32 + 16, 16)]

                def inner(k, accs):
                    p0, p1 = accs
                    r = k * 16 + b
                    w = wsplat(s1w, q * 512 + r)
                    p0 = p0 + w * rows[r, pl.ds(0, 16)]
                    p1 = p1 + w * rows[r, pl.ds(16, 16)]
                    return (p0, p1)
                a0, a1 = lax.fori_loop(0, 32, inner, (a0, a1))
                z0buf[pl.ds(b * 32, 16)] = a0
                z0buf[pl.ds(b * 32 + 16, 16)] = a1
                return _
            lax.fori_loop(0, 16, body, None)

        s1fire(0, s1rowsA, semA)
        s1fire(1, s1rowsB, semB)
        s1drain(s1rowsA, semA)
        s1acc(0, s1rowsA)
        s1fire(2, s1rowsA, semA)
        s1drain(s1rowsB, semB)
        s1acc(1, s1rowsB)
        s1fire(3, s1rowsB, semB)
        s1drain(s1rowsA, semA)
        s1acc(2, s1rowsA)
        s1drain(s1rowsB, semB)
        s1acc(3, s1rowsB)

        def s2gen(pz, _):
            z1 = plsc.load_gather(z0buf, [lanes * 32 + 2 * pz])
            z2 = plsc.load_gather(z0buf, [lanes * 32 + 2 * pz + 1])
            iz1, dz1 = interp(z1)
            iz2_unused, dz2 = interp(z2)
            del iz2_unused
            jzbuf[pz, :] = iz1
            f = 1.0 - dz2
            cz0buf[pz, :] = (1.0 - dz1) * f
            cz1buf[pz, :] = dz1 * f
            return _
        lax.fori_loop(0, P_H, s2gen, None)

        def s3genfire(blk, idxr, wr, rows, sem):
            px = lax.shift_right_logical(blk, 1) & (P_IN - 1)
            half = blk & 1
            ia = idx1buf[px, :]
            a0 = s1w[pl.ds(px * 64, 16)]
            a1 = s1w[pl.ds(px * 64 + 32, 16)]
            pbase = px * T1_PX + ia * 33

            def gen(pzl, _):
                pz = half * 8 + pzl
                jz = jzbuf[pz, :]
                c0 = cz0buf[pz, :]
                c1 = cz1buf[pz, :]
                base = pbase + pz * T1_PZ + jz
                o = pzl * 64
                idxr[pl.ds(o, 16)] = base
                idxr[pl.ds(o + 16, 16)] = base + 1
                idxr[pl.ds(o + 32, 16)] = base + 33
                idxr[pl.ds(o + 48, 16)] = base + 34
                wr[pl.ds(o, 16)] = a0 * c0
                wr[pl.ds(o + 16, 16)] = a0 * c1
                wr[pl.ds(o + 32, 16)] = a1 * c0
                wr[pl.ds(o + 48, 16)] = a1 * c1
                return _
            lax.fori_loop(0, 8, gen, None)
            for j in range(4):
                pltpu.async_copy(
                    t1_hbm.at[idxr.at[pl.ds(j * 128, 128)]],
                    rows.at[pl.ds(j * 128, 128), :], sem)

        def s3drain(idxr, rows, sem):
            for j in range(4):
                pltpu.make_async_copy(
                    t1_hbm.at[idxr.at[pl.ds(j * 128, 128)]],
                    rows.at[pl.ds(j * 128, 128), :], sem).wait()

        def s3fma(wr, rows):
            def body(b, _):
                acc = [outacc[b, pl.ds(q * 16, 16)] for q in range(4)]
                for m in range(32):
                    row = m * 16 + b
                    w = wsplat(wr, row)
                    pa = plsc.bitcast(rows[row, pl.ds(0, 32)], jnp.int32)
                    pb = plsc.bitcast(rows[row, pl.ds(32, 32)], jnp.int32)
                    acc[0] = acc[0] + w * plsc.bitcast(lax.shift_left(pa, 16), jnp.float32)
                    acc[1] = acc[1] + w * plsc.bitcast(pa, jnp.float32)
                    acc[2] = acc[2] + w * plsc.bitcast(lax.shift_left(pb, 16), jnp.float32)
                    acc[3] = acc[3] + w * plsc.bitcast(pb, jnp.float32)
                for q in range(4):
                    outacc[b, pl.ds(q * 16, 16)] = acc[q]
                return _
            lax.fori_loop(0, 16, body, None)

        s3genfire(jnp.int32(0), s3idxA, s3wA, s3rowsA, semA)

        def s3pair(g, _):
            blk = g * 2
            s3genfire(blk + 1, s3idxB, s3wB, s3rowsB, semB)
            s3drain(s3idxA, s3rowsA, semA)
            s3fma(s3wA, s3rowsA)
            s3genfire(blk + 2, s3idxA, s3wA, s3rowsA, semA)
            s3drain(s3idxB, s3rowsB, semB)
            s3fma(s3wB, s3rowsB)
            return _
        lax.fori_loop(0, P_IN, s3pair, None)
        s3drain(s3idxA, s3rowsA, semA)

        def fixup(b, _):
            for q in range(4):
                pos = b * 64 + (q // 2) * 32 + (q & 1) + 2 * lanes
                plsc.store_scatter(outfix, [pos], outacc[b, pl.ds(q * 16, 16)])
            return _
        lax.fori_loop(0, 16, fixup, None)
        pltpu.sync_copy(outfix, out_hbm.at[pl.ds((wid * BPW + b0) * 64, 1024)])
        return _
    lax.fori_loop(0, NCHUNK, chunk_body, None)


@jax.jit
def _run(x, t0, t1, bt):
    mesh = plsc.VectorSubcoreMesh(core_axis_name="c", subcore_axis_name="s")
    f = functools.partial(
        pl.kernel,
        out_type=jax.ShapeDtypeStruct((BATCH * OUTPUT_DIM,), jnp.float32),
        mesh=mesh,
        compiler_params=pltpu.CompilerParams(
            needs_layout_passes=False, use_tc_tiling_on_sc=False),
        scratch_types=[
            pltpu.VMEM((INPUT_DIM, BPW), jnp.float32),
            pltpu.VMEM((96,), jnp.float32),
            pltpu.VMEM((2048,), jnp.int32),
            pltpu.VMEM((2048,), jnp.float32),
            pltpu.VMEM((P_IN, 16), jnp.int32),
            pltpu.VMEM((512, HIDDEN_DIM), jnp.float32),
            pltpu.VMEM((512, HIDDEN_DIM), jnp.float32),
            pltpu.VMEM((16 * HIDDEN_DIM,), jnp.float32),
            pltpu.VMEM((P_H, 16), jnp.int32),
            pltpu.VMEM((P_H, 16), jnp.float32),
            pltpu.VMEM((P_H, 16), jnp.float32),
            pltpu.VMEM((512,), jnp.int32),
            pltpu.VMEM((512,), jnp.int32),
            pltpu.VMEM((512,), jnp.float32),
            pltpu.VMEM((512,), jnp.float32),
            pltpu.VMEM((512, OUTPUT_DIM), jnp.bfloat16),
            pltpu.VMEM((512, OUTPUT_DIM), jnp.bfloat16),
            pltpu.VMEM((16, OUTPUT_DIM), jnp.float32),
            pltpu.VMEM((16 * OUTPUT_DIM,), jnp.float32),
            pltpu.SemaphoreType.DMA,
            pltpu.SemaphoreType.DMA,
        ],
    )(_sc_body)
    return f(x, t0, t1, bt)


def kernel(x, level0_params, level1_params):
    t0 = jnp.transpose(level0_params, (3, 0, 1, 2)).reshape(P_IN * T0_PX, HIDDEN_DIM)
    t1 = jnp.transpose(level1_params.astype(jnp.bfloat16),
                       (3, 4, 0, 1, 2)).reshape(P_IN * T1_PX, OUTPUT_DIM)
    bt = jnp.asarray(_BT)
    out_t = _run(x, t0, t1, bt)
    return out_t.reshape(BATCH, OUTPUT_DIM).T
